# baseline (speedup 1.0000x reference)
import jax
import jax.numpy as jnp
from jax import lax
from jax.experimental import pallas as pl
from jax.experimental.pallas import tpu as pltpu
from jax.experimental.pallas import tpu_sc as plsc

B, S, D = 4, 8192, 768
NC, NS = 2, 16
NW = NC * NS
SPW = S // NW
SCHUNK = 32
NSC = SPW // SCHUNK
NTASK = NSC * B
LANES = 16
NBUF = 4


def _sc_body(x_hbm, pos_hbm, out_hbm, posbuf, xb0, xb1, xb2, xb3,
             sp, si0, si1, si2, si3, so0, so1, so2, so3):
    wid = lax.axis_index("s") * NC + lax.axis_index("c")
    even = wid % 2 == 0
    s0 = wid * SPW
    xb = (xb0, xb1, xb2, xb3)
    si = (si0, si1, si2, si3)
    so = (so0, so1, so2, so3)

    def xrow(t):
        sc, b = divmod(t, B)
        return b * S + s0 + sc * SCHUNK

    def pos_slice(sc):
        return pos_hbm.at[pl.ds(s0 + sc * SCHUNK, SCHUNK)]

    def issue_in(t):
        r = t % NBUF
        pltpu.async_copy(x_hbm.at[pl.ds(xrow(t), SCHUNK)], xb[r], si[r])

    def issue_out(t):
        q = t % NBUF
        pltpu.async_copy(xb[q], out_hbm.at[pl.ds(xrow(t), SCHUNK)], so[q])

    def wait_in(t):
        r = t % NBUF
        pltpu.make_async_copy(
            x_hbm.at[pl.ds(xrow(t), SCHUNK)], xb[r], si[r]).wait()

    def wait_out(t):
        q = t % NBUF
        pltpu.make_async_copy(
            xb[q], out_hbm.at[pl.ds(xrow(t), SCHUNK)], so[q]).wait()

    pin = pltpu.async_copy(pos_slice(0), posbuf, sp)
    issue_in(0)
    issue_in(1)

    for t in range(NTASK):
        sc, b = divmod(t, B)
        p = t % NBUF
        if b == 0:
            pin.wait()
        wait_in(t)
        if t - 2 >= 0 and t + 2 < NTASK:
            wait_out(t - 2)
        if t >= 1 and t + 2 < NTASK:
            @pl.when(even)
            def _():
                issue_out(t - 1)
                issue_in(t + 2)

            @pl.when(jnp.logical_not(even))
            def _():
                issue_in(t + 2)
                issue_out(t - 1)
        elif t >= 1:
            issue_out(t - 1)
        elif t + 2 < NTASK:
            issue_in(t + 2)

        @plsc.parallel_loop(0, SCHUNK, step=1)
        def _(rr):
            @plsc.parallel_loop(0, D, step=LANES, unroll=8)
            def _(c):
                plsc.addupdate(xb[p].at[rr].at[pl.ds(c, LANES)],
                               posbuf.at[rr][pl.ds(c, LANES)])

        if b == B - 1 and sc + 1 < NSC:
            pin = pltpu.async_copy(pos_slice(sc + 1), posbuf, sp)

    issue_out(NTASK - 1)
    for t in (NTASK - 4, NTASK - 3, NTASK - 2, NTASK - 1):
        wait_out(t)


_sc_call = pl.kernel(
    _sc_body,
    out_type=jax.ShapeDtypeStruct((B * S, D), jnp.float32),
    mesh=plsc.VectorSubcoreMesh(core_axis_name="c", subcore_axis_name="s"),
    scratch_types=[
        pltpu.VMEM((SCHUNK, D), jnp.float32),
        pltpu.VMEM((SCHUNK, D), jnp.float32),
        pltpu.VMEM((SCHUNK, D), jnp.float32),
        pltpu.VMEM((SCHUNK, D), jnp.float32),
        pltpu.VMEM((SCHUNK, D), jnp.float32),
        pltpu.SemaphoreType.DMA,
        pltpu.SemaphoreType.DMA,
        pltpu.SemaphoreType.DMA,
        pltpu.SemaphoreType.DMA,
        pltpu.SemaphoreType.DMA,
        pltpu.SemaphoreType.DMA,
        pltpu.SemaphoreType.DMA,
        pltpu.SemaphoreType.DMA,
        pltpu.SemaphoreType.DMA,
    ],
)


def kernel(x, pos_table):
    out = _sc_call(x.reshape(B * S, D), pos_table)
    return out.reshape(B, S, D)


# speedup vs baseline: 1.0025x; 1.0025x over previous
import jax
import jax.numpy as jnp
from jax import lax
from jax.experimental import pallas as pl
from jax.experimental.pallas import tpu as pltpu
from jax.experimental.pallas import tpu_sc as plsc

B, S, D = 4, 8192, 768
NC, NS = 2, 16
NW = NC * NS
SPW = S // NW
SCHUNK = 32
NSC = SPW // SCHUNK
NTASK = NSC * B
LANES = 16
NBUF = 4


def _sc_body(x_hbm, pos_hbm, out_hbm, posbuf, xb0, xb1, xb2, xb3,
             sp, si0, si1, si2, si3, so0, so1, so2, so3):
    wid = lax.axis_index("s") * NC + lax.axis_index("c")
    s0 = wid * SPW
    xb = (xb0, xb1, xb2, xb3)
    si = (si0, si1, si2, si3)
    so = (so0, so1, so2, so3)

    def xrow(t):
        sc, b = divmod(t, B)
        return b * S + s0 + sc * SCHUNK

    def pos_slice(sc):
        return pos_hbm.at[pl.ds(s0 + sc * SCHUNK, SCHUNK)]

    pin = pltpu.async_copy(pos_slice(0), posbuf, sp)
    xin = [None] * NBUF
    xout = [None] * NBUF
    xin[0] = pltpu.async_copy(x_hbm.at[pl.ds(xrow(0), SCHUNK)], xb[0], si[0])
    xin[1] = pltpu.async_copy(x_hbm.at[pl.ds(xrow(1), SCHUNK)], xb[1], si[1])

    for t in range(NTASK):
        sc, b = divmod(t, B)
        p = t % NBUF
        if b == 0:
            pin.wait()
        xin[p].wait()
        if t >= 1:
            q = (t - 1) % NBUF
            xout[q] = pltpu.async_copy(
                xb[q], out_hbm.at[pl.ds(xrow(t - 1), SCHUNK)], so[q])
        if t + 2 < NTASK:
            r = (t + 2) % NBUF
            if xout[r] is not None:
                xout[r].wait()
            xin[r] = pltpu.async_copy(
                x_hbm.at[pl.ds(xrow(t + 2), SCHUNK)], xb[r], si[r])

        @plsc.parallel_loop(0, SCHUNK, step=1)
        def _(rr):
            @plsc.parallel_loop(0, D, step=LANES, unroll=8)
            def _(c):
                plsc.addupdate(xb[p].at[rr].at[pl.ds(c, LANES)],
                               posbuf.at[rr][pl.ds(c, LANES)])

        if b == B - 1 and sc + 1 < NSC:
            pin = pltpu.async_copy(pos_slice(sc + 1), posbuf, sp)

    last = (NTASK - 1) % NBUF
    xout[last] = pltpu.async_copy(
        xb[last], out_hbm.at[pl.ds(xrow(NTASK - 1), SCHUNK)], so[last])
    for q in range(NBUF):
        if xout[q] is not None:
            xout[q].wait()


_sc_call = pl.kernel(
    _sc_body,
    out_type=jax.ShapeDtypeStruct((B * S, D), jnp.float32),
    mesh=plsc.VectorSubcoreMesh(core_axis_name="c", subcore_axis_name="s"),
    scratch_types=[
        pltpu.VMEM((SCHUNK, D), jnp.float32),
        pltpu.VMEM((SCHUNK, D), jnp.float32),
        pltpu.VMEM((SCHUNK, D), jnp.float32),
        pltpu.VMEM((SCHUNK, D), jnp.float32),
        pltpu.VMEM((SCHUNK, D), jnp.float32),
        pltpu.SemaphoreType.DMA,
        pltpu.SemaphoreType.DMA,
        pltpu.SemaphoreType.DMA,
        pltpu.SemaphoreType.DMA,
        pltpu.SemaphoreType.DMA,
        pltpu.SemaphoreType.DMA,
        pltpu.SemaphoreType.DMA,
        pltpu.SemaphoreType.DMA,
        pltpu.SemaphoreType.DMA,
    ],
)


def kernel(x, pos_table):
    out = _sc_call(x.reshape(B * S, D), pos_table)
    return out.reshape(B, S, D)


# speedup vs baseline: 1.0346x; 1.0320x over previous
import jax
import jax.numpy as jnp
from jax import lax
from jax.experimental import pallas as pl
from jax.experimental.pallas import tpu as pltpu
from jax.experimental.pallas import tpu_sc as plsc

B, S, D = 4, 8192, 768
NC, NS = 2, 16
NW = NC * NS
SPW = S // NW
SCHUNK = 16
NSC = SPW // SCHUNK
NTASK = NSC * B
LANES = 16
NBUF = 6


def _sc_body(x_hbm, pos_hbm, out_hbm, pos0, pos1,
             xb0, xb1, xb2, xb3, xb4, xb5,
             sp0, sp1, si0, si1, si2, si3, si4, si5,
             so0, so1, so2, so3, so4, so5):
    wid = lax.axis_index("s") * NC + lax.axis_index("c")
    s0 = wid * SPW
    posb = (pos0, pos1)
    sp = (sp0, sp1)
    xb = (xb0, xb1, xb2, xb3, xb4, xb5)
    si = (si0, si1, si2, si3, si4, si5)
    so = (so0, so1, so2, so3, so4, so5)

    def xrow(t):
        sc, b = divmod(t, B)
        return b * S + s0 + sc * SCHUNK

    def pos_slice(sc):
        return pos_hbm.at[pl.ds(s0 + sc * SCHUNK, SCHUNK)]

    pin = [pltpu.async_copy(pos_slice(0), pos0, sp0),
           pltpu.async_copy(pos_slice(1), pos1, sp1)]
    xin = [None] * NBUF
    xout = [None] * NBUF
    xin[0] = pltpu.async_copy(x_hbm.at[pl.ds(xrow(0), SCHUNK)], xb[0], si[0])
    xin[1] = pltpu.async_copy(x_hbm.at[pl.ds(xrow(1), SCHUNK)], xb[1], si[1])

    for t in range(NTASK):
        sc, b = divmod(t, B)
        p = t % NBUF
        pc = sc & 1
        if b == 0:
            pin[pc].wait()
        xin[p].wait()
        if t >= 1:
            q = (t - 1) % NBUF
            xout[q] = pltpu.async_copy(
                xb[q], out_hbm.at[pl.ds(xrow(t - 1), SCHUNK)], so[q])
        if t + 2 < NTASK:
            r = (t + 2) % NBUF
            if xout[r] is not None:
                xout[r].wait()
            xin[r] = pltpu.async_copy(
                x_hbm.at[pl.ds(xrow(t + 2), SCHUNK)], xb[r], si[r])

        @plsc.parallel_loop(0, SCHUNK, step=1)
        def _(rr):
            @plsc.parallel_loop(0, D, step=LANES, unroll=8)
            def _(c):
                plsc.addupdate(xb[p].at[rr].at[pl.ds(c, LANES)],
                               posb[pc].at[rr][pl.ds(c, LANES)])

        if b == B - 1 and sc + 2 < NSC:
            pin[pc] = pltpu.async_copy(pos_slice(sc + 2), posb[pc], sp[pc])

    last = (NTASK - 1) % NBUF
    xout[last] = pltpu.async_copy(
        xb[last], out_hbm.at[pl.ds(xrow(NTASK - 1), SCHUNK)], so[last])
    for q in range(NBUF):
        if xout[q] is not None:
            xout[q].wait()


_sc_call = pl.kernel(
    _sc_body,
    out_type=jax.ShapeDtypeStruct((B * S, D), jnp.float32),
    mesh=plsc.VectorSubcoreMesh(core_axis_name="c", subcore_axis_name="s"),
    scratch_types=(
        [pltpu.VMEM((SCHUNK, D), jnp.float32)] * 2
        + [pltpu.VMEM((SCHUNK, D), jnp.float32)] * NBUF
        + [pltpu.SemaphoreType.DMA] * (2 + 2 * NBUF)
    ),
)


def kernel(x, pos_table):
    out = _sc_call(x.reshape(B * S, D), pos_table)
    return out.reshape(B, S, D)
